# baseline (speedup 1.0000x reference)
import jax
import jax.numpy as jnp
from jax import lax
from jax.experimental import pallas as pl
from jax.experimental.pallas import tpu as pltpu
from jax.experimental.pallas import tpu_sc as plsc

SPATIAL = 64
C = 32
NV = 2 * SPATIAL ** 3
NC = 2
NS = 16
LANES = 16

W = 53248
NWIN = 10
NPASS = 5
TAIL_W = NV - (NWIN - 1) * W
TRASH = W
SH_ROWS = W + 128

STRIPE = W // NS
TAIL_STRIPE = TAIL_W // NS
BLK = 128
SEG = 3136

N_POINTS = 200000
NSL = -(-N_POINTS // (NS * BLK)) * BLK
N_PAD = NSL * NS
NBLK = NSL // BLK


def _sc_body(b_hbm, x_hbm, y_hbm, z_hbm, feats_hbm, out_hbm,
             flat_v, stg_v, dstc0_v, dstc1_v, fbuf0_v, fbuf1_v,
             shared, sem0, sem1, semd, semc):
    c = lax.axis_index("c")
    s = lax.axis_index("s")
    sbase = s * NSL

    for d, src in enumerate((b_hbm, x_hbm, y_hbm, z_hbm)):
        for t in range(NSL // SEG):
            pltpu.sync_copy(src.at[pl.ds(sbase + t * SEG, SEG)], stg_v)

            def fb(i, carry):
                sl = pl.ds(t * SEG + i * LANES, LANES)
                cv = stg_v[pl.ds(i * LANES, LANES)]
                if d == 0:
                    flat_v[sl] = cv
                else:
                    flat_v[sl] = flat_v[sl] * SPATIAL + cv
                return carry

            lax.fori_loop(0, SEG // LANES, fb, 0)

    zf = jnp.zeros((LANES,), jnp.float32)

    def build_dst(dstc, blk, lo):
        base = blk * BLK
        for k in range(BLK // LANES):
            v = flat_v[pl.ds(base + k * LANES, LANES)]
            m = (v >= lo) & (v < lo + W)
            trash = (TRASH + k * LANES) + lax.iota(jnp.int32, LANES)
            dstc[pl.ds(k * LANES, LANES)] = jnp.where(m, v - lo, trash)

    def stream(fbuf, blk, sem):
        pltpu.async_copy(
            feats_hbm.at[pl.ds(sbase + blk * BLK, BLK)], fbuf, sem)

    def swait(fbuf, blk, sem):
        pltpu.make_async_copy(
            feats_hbm.at[pl.ds(sbase + blk * BLK, BLK)], fbuf, sem).wait()

    def zslice(t):
        return shared.at[pl.ds(s * STRIPE + t * BLK, BLK)]

    def pass_body(p, carry):
        wid = p * NC + c
        lo = wid * W

        def zb(i, carry2):
            fbuf0_v[i, pl.ds(0, LANES)] = zf
            fbuf0_v[i, pl.ds(LANES, LANES)] = zf
            return carry2

        lax.fori_loop(0, BLK, zb, 0)

        @pl.when(p > 0)
        def _wait_drain():
            prev_lo = (p - 1) * NC * W + c * W
            pltpu.make_async_copy(
                shared.at[pl.ds(s * STRIPE, STRIPE)],
                out_hbm.at[pl.ds(prev_lo + s * STRIPE, STRIPE)],
                semd).wait()

        for t in range(STRIPE // BLK):
            pltpu.async_copy(fbuf0_v, zslice(t), semd)
        stream(fbuf1_v, jnp.int32(1), sem1)
        for t in range(STRIPE // BLK):
            pltpu.make_async_copy(fbuf0_v, zslice(t), semd).wait()

        plsc.subcore_barrier()

        stream(fbuf0_v, jnp.int32(0), sem0)

        def hb(bb, carry3):
            b0 = 2 * bb
            b1 = 2 * bb + 1
            build_dst(dstc0_v, b0, lo)
            swait(fbuf0_v, b0, sem0)
            pltpu.async_copy(fbuf0_v, shared.at[dstc0_v], semd, add=True)
            build_dst(dstc1_v, b1, lo)
            swait(fbuf1_v, b1, sem1)
            pltpu.async_copy(fbuf1_v, shared.at[dstc1_v], semc, add=True)
            pltpu.make_async_copy(fbuf0_v, shared.at[dstc0_v], semd).wait()
            stream(fbuf0_v, b0 + 2, sem0)
            pltpu.make_async_copy(fbuf1_v, shared.at[dstc1_v], semc).wait()
            stream(fbuf1_v, b1 + 2, sem1)
            return carry3

        lax.fori_loop(0, NBLK // 2 - 1, hb, 0)

        build_dst(dstc0_v, jnp.int32(NBLK - 2), lo)
        swait(fbuf0_v, jnp.int32(NBLK - 2), sem0)
        pltpu.sync_copy(fbuf0_v, shared.at[dstc0_v], add=True)
        build_dst(dstc1_v, jnp.int32(NBLK - 1), lo)
        swait(fbuf1_v, jnp.int32(NBLK - 1), sem1)
        pltpu.sync_copy(fbuf1_v, shared.at[dstc1_v], add=True)

        plsc.subcore_barrier()

        full = lo + W <= NV

        @pl.when(full)
        def _drain_full():
            pltpu.async_copy(shared.at[pl.ds(s * STRIPE, STRIPE)],
                             out_hbm.at[pl.ds(lo + s * STRIPE, STRIPE)],
                             semd)

        @pl.when(jnp.logical_not(full))
        def _drain_tail():
            pltpu.sync_copy(
                shared.at[pl.ds(s * TAIL_STRIPE, TAIL_STRIPE)],
                out_hbm.at[pl.ds(lo + s * TAIL_STRIPE, TAIL_STRIPE)])

        return carry

    lax.fori_loop(0, NPASS, pass_body, 0)

    @pl.when(c == 0)
    def _final_wait():
        last_lo = (NPASS - 1) * NC * W
        pltpu.make_async_copy(
            shared.at[pl.ds(s * STRIPE, STRIPE)],
            out_hbm.at[pl.ds(last_lo + s * STRIPE, STRIPE)],
            semd).wait()


def kernel(coords, features, batch_idx, batch_size):
    n = coords.shape[0]
    shift = jnp.asarray(batch_size, jnp.int32) - 2
    pad = N_PAD - n
    b_a = jnp.pad(batch_idx.astype(jnp.int32), (0, pad), constant_values=-1)
    x_a = jnp.pad(coords[:, 0].astype(jnp.int32), (0, pad),
                  constant_values=-1)
    y_a = jnp.pad(coords[:, 1].astype(jnp.int32), (0, pad),
                  constant_values=-1)
    z_a = jnp.pad(coords[:, 2].astype(jnp.int32) + shift, (0, pad),
                  constant_values=-1)
    feats = jnp.pad(features.astype(jnp.float32), ((0, pad), (0, 0)))

    mesh = plsc.VectorSubcoreMesh(core_axis_name="c", subcore_axis_name="s",
                                  num_cores=NC, num_subcores=NS)
    run = pl.kernel(
        _sc_body,
        out_type=jax.ShapeDtypeStruct((NV, C), jnp.float32),
        mesh=mesh,
        scratch_types=[
            pltpu.VMEM((NSL,), jnp.int32),
            pltpu.VMEM((SEG,), jnp.int32),
            pltpu.VMEM((BLK,), jnp.int32),
            pltpu.VMEM((BLK,), jnp.int32),
            pltpu.VMEM((BLK, C), jnp.float32),
            pltpu.VMEM((BLK, C), jnp.float32),
            pltpu.VMEM_SHARED((SH_ROWS, C), jnp.float32),
            pltpu.SemaphoreType.DMA,
            pltpu.SemaphoreType.DMA,
            pltpu.SemaphoreType.DMA,
            pltpu.SemaphoreType.DMA,
        ],
        compiler_params=pltpu.CompilerParams(needs_layout_passes=False,
                                             use_tc_tiling_on_sc=False),
    )
    return run(b_a, x_a, y_a, z_a, feats)


# speedup vs baseline: 1.0361x; 1.0361x over previous
import jax
import jax.numpy as jnp
from jax import lax
from jax.experimental import pallas as pl
from jax.experimental.pallas import tpu as pltpu
from jax.experimental.pallas import tpu_sc as plsc

SPATIAL = 64
C = 32
NV = 2 * SPATIAL ** 3
NC = 2
NS = 16
LANES = 16

W = 53248
NWIN = 10
NPASS = 5
TAIL_W = NV - (NWIN - 1) * W
TRASH = W
SH_ROWS = W + 128

STRIPE = W // NS
TAIL_STRIPE = TAIL_W // NS
BLK = 128
SEG = 3136

N_POINTS = 200000
NSL = -(-N_POINTS // (NS * BLK)) * BLK
N_PAD = NSL * NS
NBLK = NSL // BLK


def _sc_body(b_hbm, x_hbm, y_hbm, z_hbm, feats_hbm, out_hbm,
             flat_v, stg_v, dstc0_v, dstc1_v, fbuf0_v, fbuf1_v,
             shared, sem0, sem1, semd):
    c = lax.axis_index("c")
    s = lax.axis_index("s")
    sbase = s * NSL

    for d, src in enumerate((b_hbm, x_hbm, y_hbm, z_hbm)):
        for t in range(NSL // SEG):
            pltpu.sync_copy(src.at[pl.ds(sbase + t * SEG, SEG)], stg_v)

            def fb(i, carry):
                sl = pl.ds(t * SEG + i * LANES, LANES)
                cv = stg_v[pl.ds(i * LANES, LANES)]
                if d == 0:
                    flat_v[sl] = cv
                else:
                    flat_v[sl] = flat_v[sl] * SPATIAL + cv
                return carry

            lax.fori_loop(0, SEG // LANES, fb, 0)

    zf = jnp.zeros((LANES,), jnp.float32)

    def build_dst(dstc, blk, lo):
        base = blk * BLK
        for k in range(BLK // LANES):
            v = flat_v[pl.ds(base + k * LANES, LANES)]
            m = (v >= lo) & (v < lo + W)
            trash = (TRASH + k * LANES) + lax.iota(jnp.int32, LANES)
            dstc[pl.ds(k * LANES, LANES)] = jnp.where(m, v - lo, trash)

    def stream(fbuf, blk, sem):
        pltpu.async_copy(
            feats_hbm.at[pl.ds(sbase + blk * BLK, BLK)], fbuf, sem)

    def swait(fbuf, blk, sem):
        pltpu.make_async_copy(
            feats_hbm.at[pl.ds(sbase + blk * BLK, BLK)], fbuf, sem).wait()

    def zslice(t):
        return shared.at[pl.ds(s * STRIPE + t * BLK, BLK)]

    def pass_body(p, carry):
        wid = p * NC + c
        lo = wid * W

        def zb(i, carry2):
            fbuf0_v[i, pl.ds(0, LANES)] = zf
            fbuf0_v[i, pl.ds(LANES, LANES)] = zf
            return carry2

        lax.fori_loop(0, BLK, zb, 0)

        @pl.when(p > 0)
        def _wait_drain():
            prev_lo = (p - 1) * NC * W + c * W
            pltpu.make_async_copy(
                shared.at[pl.ds(s * STRIPE, STRIPE)],
                out_hbm.at[pl.ds(prev_lo + s * STRIPE, STRIPE)],
                semd).wait()

        for t in range(STRIPE // BLK):
            pltpu.async_copy(fbuf0_v, zslice(t), semd)
        stream(fbuf1_v, jnp.int32(1), sem1)
        for t in range(STRIPE // BLK):
            pltpu.make_async_copy(fbuf0_v, zslice(t), semd).wait()

        plsc.subcore_barrier()

        stream(fbuf0_v, jnp.int32(0), sem0)

        def hb(bb, carry3):
            b0 = 2 * bb
            b1 = 2 * bb + 1
            build_dst(dstc0_v, b0, lo)
            swait(fbuf0_v, b0, sem0)
            pltpu.sync_copy(fbuf0_v, shared.at[dstc0_v], add=True)
            stream(fbuf0_v, b0 + 2, sem0)
            build_dst(dstc1_v, b1, lo)
            swait(fbuf1_v, b1, sem1)
            pltpu.sync_copy(fbuf1_v, shared.at[dstc1_v], add=True)
            stream(fbuf1_v, b1 + 2, sem1)
            return carry3

        lax.fori_loop(0, NBLK // 2 - 1, hb, 0)

        build_dst(dstc0_v, jnp.int32(NBLK - 2), lo)
        swait(fbuf0_v, jnp.int32(NBLK - 2), sem0)
        pltpu.sync_copy(fbuf0_v, shared.at[dstc0_v], add=True)
        build_dst(dstc1_v, jnp.int32(NBLK - 1), lo)
        swait(fbuf1_v, jnp.int32(NBLK - 1), sem1)
        pltpu.sync_copy(fbuf1_v, shared.at[dstc1_v], add=True)

        plsc.subcore_barrier()

        full = lo + W <= NV

        @pl.when(full)
        def _drain_full():
            pltpu.async_copy(shared.at[pl.ds(s * STRIPE, STRIPE)],
                             out_hbm.at[pl.ds(lo + s * STRIPE, STRIPE)],
                             semd)

        @pl.when(jnp.logical_not(full))
        def _drain_tail():
            pltpu.sync_copy(
                shared.at[pl.ds(s * TAIL_STRIPE, TAIL_STRIPE)],
                out_hbm.at[pl.ds(lo + s * TAIL_STRIPE, TAIL_STRIPE)])

        return carry

    lax.fori_loop(0, NPASS, pass_body, 0)

    @pl.when(c == 0)
    def _final_wait():
        last_lo = (NPASS - 1) * NC * W
        pltpu.make_async_copy(
            shared.at[pl.ds(s * STRIPE, STRIPE)],
            out_hbm.at[pl.ds(last_lo + s * STRIPE, STRIPE)],
            semd).wait()


def kernel(coords, features, batch_idx, batch_size):
    n = coords.shape[0]
    shift = jnp.asarray(batch_size, jnp.int32) - 2
    pad = N_PAD - n
    b_a = jnp.pad(batch_idx.astype(jnp.int32), (0, pad), constant_values=-1)
    x_a = jnp.pad(coords[:, 0].astype(jnp.int32), (0, pad),
                  constant_values=-1)
    y_a = jnp.pad(coords[:, 1].astype(jnp.int32), (0, pad),
                  constant_values=-1)
    z_a = jnp.pad(coords[:, 2].astype(jnp.int32) + shift, (0, pad),
                  constant_values=-1)
    feats = jnp.pad(features.astype(jnp.float32), ((0, pad), (0, 0)))

    mesh = plsc.VectorSubcoreMesh(core_axis_name="c", subcore_axis_name="s",
                                  num_cores=NC, num_subcores=NS)
    run = pl.kernel(
        _sc_body,
        out_type=jax.ShapeDtypeStruct((NV, C), jnp.float32),
        mesh=mesh,
        scratch_types=[
            pltpu.VMEM((NSL,), jnp.int32),
            pltpu.VMEM((SEG,), jnp.int32),
            pltpu.VMEM((BLK,), jnp.int32),
            pltpu.VMEM((BLK,), jnp.int32),
            pltpu.VMEM((BLK, C), jnp.float32),
            pltpu.VMEM((BLK, C), jnp.float32),
            pltpu.VMEM_SHARED((SH_ROWS, C), jnp.float32),
            pltpu.SemaphoreType.DMA,
            pltpu.SemaphoreType.DMA,
            pltpu.SemaphoreType.DMA,
        ],
        compiler_params=pltpu.CompilerParams(needs_layout_passes=False,
                                             use_tc_tiling_on_sc=False),
    )
    return run(b_a, x_a, y_a, z_a, feats)
